# baseline (speedup 1.0000x reference)
import functools

import numpy as np
import jax
import jax.numpy as jnp
from jax.experimental import pallas as pl
from jax.experimental.pallas import tpu as pltpu

FS = 16000
M = 800
K = 1024
HOP = 600
NBINS = 256
ORIG_L = 13
MIN_L = 3
MIN_DB = 10.0
DEFAULT_RT = 0.5


def _dft_mats():
    k = np.arange(K, dtype=np.float64)
    w = 0.5 - 0.5 * np.cos(2.0 * np.pi * np.arange(M, dtype=np.float64) / M)
    wf = np.zeros(K, dtype=np.float64)
    lp = (K - M) // 2
    wf[lp:lp + M] = w
    f = np.arange(NBINS, dtype=np.float64)
    ang = 2.0 * np.pi * np.outer(k, f) / K
    cs = np.concatenate([np.cos(ang) * wf[:, None],
                         np.sin(ang) * wf[:, None]], axis=1)
    return jnp.asarray(cs.astype(np.float32))


def _body(nf, amid_ref, edge_ref, w1_ref, w2_ref, pc_ref, o_ref):
    amid = amid_ref[0]
    row0 = edge_ref[0, 0:1, :]
    rowl = edge_ref[0, 1:2, :]
    a0 = jnp.concatenate([row0, amid], axis=0)
    a1 = jnp.concatenate([amid, rowl], axis=0)[:, 0:K - HOP]
    cs = (jnp.dot(a0, w1_ref[...], preferred_element_type=jnp.float32) +
          jnp.dot(a1, w2_ref[...], preferred_element_type=jnp.float32))
    c = cs[:, :NBINS]
    s = cs[:, NBINS:]
    P = c * c + s * s

    Pp = jnp.concatenate(
        [P, jnp.full((ORIG_L, NBINS), jnp.inf, jnp.float32)], axis=0)
    sh = [Pp[t:t + nf, :] for t in range(ORIG_L + 1)]

    dsh = [sh[t + 1] < sh[t] for t in range(ORIG_L - 1)]
    d = dsh[0]

    rl = d.astype(jnp.int32)
    am = d
    for m in range(2, ORIG_L):
        am = am & dsh[m - 1]
        rl = rl + am.astype(jnp.int32)

    maxrl = jnp.max(rl, axis=0, keepdims=True)
    lstar = jnp.where(maxrl >= MIN_L - 1,
                      jnp.minimum(maxrl + 1, ORIG_L), 0)
    valid = (rl + 1 >= lstar) & (lstar >= MIN_L)

    lf = lstar.astype(jnp.float32)
    xm = (lf - 1.0) * 0.5
    den = lf * (lf * lf - 1.0) / 12.0
    acc = jnp.zeros((nf, NBINS), jnp.float32)
    num = jnp.zeros((nf, NBINS), jnp.float32)
    llast = jnp.zeros((nf, NBINS), jnp.float32)
    ldb = None
    for t in range(ORIG_L - 1, -1, -1):
        tm = t < lstar
        acc = acc + jnp.where(tm, sh[t], 0.0)
        ldb = 10.0 * jnp.log10(acc)
        num = num + jnp.where(tm, (t - xm) * ldb, 0.0)
        llast = jnp.where(lstar == t + 1, ldb, llast)
    slope = num / den
    rt = -60.0 / slope * HOP / FS

    mk = valid & (llast - ldb < -MIN_DB)

    rtb = jax.lax.bitcast_convert_type(rt, jnp.int32)
    rtbs = jnp.where(mk, rtb, jnp.int32(0x7FFFFFFF))
    kcnt = jnp.sum(mk.astype(jnp.int32))
    rlo = jnp.maximum((kcnt - 1) // 2, 0)
    rhi = kcnt // 2

    ones_row = jnp.ones((1, nf), jnp.float32)
    target = (rhi + 1).astype(jnp.float32)

    def search_step(_, carry):
        lo, hi = carry
        mid = (lo + hi) // 2
        cmpf = jnp.where(rtbs <= mid, 1.0, 0.0)
        cnt = jnp.sum(jnp.dot(ones_row, cmpf,
                              preferred_element_type=jnp.float32))
        g = cnt >= target
        return jnp.where(g, lo, mid + 1), jnp.where(g, mid, hi)

    theta, _ = jax.lax.fori_loop(
        0, 31, search_step, (jnp.int32(0), jnp.int32(0x7F800000)))
    below = rtbs < theta
    clt = jnp.sum(below.astype(jnp.int32))
    pred = jnp.max(jnp.where(below, rtbs, 0))
    vlo_bits = jnp.where(clt > rlo, pred, theta)
    vlo = jax.lax.bitcast_convert_type(vlo_bits, jnp.float32)
    vhi = jax.lax.bitcast_convert_type(theta, jnp.float32)
    med = (vlo + vhi) * jnp.float32(0.5)

    out = pc_ref[0, 0] + pc_ref[0, 1] * med
    out = jnp.where(kcnt > 0, out, jnp.nan)
    out = jnp.where(jnp.isnan(out), jnp.float32(DEFAULT_RT), out)
    o_ref[...] = jnp.maximum(out, jnp.float32(0.01)).reshape(1, 1, 1)


def kernel(y, poly_coeffs):
    b = y.shape[0]
    t_len = y.shape[-1]
    nf = 1 + t_len // HOP
    pad = K // 2
    y2 = y[:, 0, :]
    off = HOP - pad
    amid = y2[:, off:off + (nf - 1) * HOP].reshape(b, nf - 1, HOP)
    row0 = jnp.concatenate(
        [y2[:, pad:0:-1], y2[:, :off]], axis=1)
    tail_direct = t_len - (nf * HOP - pad)
    npad_r = HOP - tail_direct
    rowl = jnp.concatenate(
        [y2[:, nf * HOP - pad:],
         y2[:, t_len - 2:t_len - 2 - npad_r:-1]], axis=1)
    edges = jnp.stack([row0, rowl], axis=1)
    cs = _dft_mats()
    w1 = cs[:HOP]
    w2 = cs[HOP:]
    pc = poly_coeffs.reshape(1, 2).astype(jnp.float32)

    out = pl.pallas_call(
        functools.partial(_body, nf),
        grid=(b,),
        in_specs=[
            pl.BlockSpec((1, nf - 1, HOP), lambda i: (i, 0, 0)),
            pl.BlockSpec((1, 2, HOP), lambda i: (i, 0, 0)),
            pl.BlockSpec((HOP, 2 * NBINS), lambda i: (0, 0)),
            pl.BlockSpec((K - HOP, 2 * NBINS), lambda i: (0, 0)),
            pl.BlockSpec((1, 2), lambda i: (0, 0)),
        ],
        out_specs=pl.BlockSpec((1, 1, 1), lambda i: (i, 0, 0)),
        out_shape=jax.ShapeDtypeStruct((b, 1, 1), jnp.float32),
        compiler_params=pltpu.CompilerParams(
            dimension_semantics=("arbitrary",)),
    )(amid, edges, w1, w2, pc)
    return out


# speedup vs baseline: 1.3223x; 1.3223x over previous
import functools

import numpy as np
import jax
import jax.numpy as jnp
from jax.experimental import pallas as pl
from jax.experimental.pallas import tpu as pltpu

FS = 16000
M = 800
K = 1024
HOP = 600
NBINS = 256
ORIG_L = 13
MIN_L = 3
MIN_DB = 10.0
DEFAULT_RT = 0.5


def _dft_mats():
    k = np.arange(K, dtype=np.float64)
    w = 0.5 - 0.5 * np.cos(2.0 * np.pi * np.arange(M, dtype=np.float64) / M)
    wf = np.zeros(K, dtype=np.float64)
    lp = (K - M) // 2
    wf[lp:lp + M] = w
    f = np.arange(NBINS, dtype=np.float64)
    ang = 2.0 * np.pi * np.outer(k, f) / K
    cs = np.concatenate([np.cos(ang) * wf[:, None],
                         np.sin(ang) * wf[:, None]], axis=1)
    return jnp.asarray(cs.astype(np.float32))


def _body(nf, amid_ref, edge_ref, w1_ref, w2_ref, pc_ref, o_ref):
    amid = amid_ref[0]
    row0 = edge_ref[0, 0:1, :]
    rowl = edge_ref[0, 1:2, :]
    a0 = jnp.concatenate([row0, amid], axis=0)
    a1 = jnp.concatenate([amid, rowl], axis=0)[:, 0:K - HOP]
    cs = (jnp.dot(a0, w1_ref[...], preferred_element_type=jnp.float32) +
          jnp.dot(a1, w2_ref[...], preferred_element_type=jnp.float32))
    c = cs[:, :NBINS]
    s = cs[:, NBINS:]
    P = c * c + s * s

    Pp = jnp.concatenate(
        [P, jnp.full((ORIG_L, NBINS), jnp.inf, jnp.float32)], axis=0)
    sh = [Pp[t:t + nf, :] for t in range(ORIG_L + 1)]

    dsh = [sh[t + 1] < sh[t] for t in range(ORIG_L - 1)]
    d = dsh[0]

    rl = d.astype(jnp.int32)
    am = d
    for m in range(2, ORIG_L):
        am = am & dsh[m - 1]
        rl = rl + am.astype(jnp.int32)

    maxrl = jnp.max(rl, axis=0, keepdims=True)
    lstar = jnp.where(maxrl >= MIN_L - 1,
                      jnp.minimum(maxrl + 1, ORIG_L), 0)
    valid = (rl + 1 >= lstar) & (lstar >= MIN_L)

    lf = lstar.astype(jnp.float32)
    xm = (lf - 1.0) * 0.5
    den = lf * (lf * lf - 1.0) / 12.0
    acc = jnp.zeros((nf, NBINS), jnp.float32)
    num = jnp.zeros((nf, NBINS), jnp.float32)
    llast = jnp.zeros((nf, NBINS), jnp.float32)
    ldb = None
    for t in range(ORIG_L - 1, -1, -1):
        tm = t < lstar
        acc = acc + jnp.where(tm, sh[t], 0.0)
        ldb = 10.0 * jnp.log10(acc)
        num = num + jnp.where(tm, (t - xm) * ldb, 0.0)
        llast = jnp.where(lstar == t + 1, ldb, llast)
    slope = num / den
    rt = -60.0 / slope * HOP / FS

    mk = valid & (llast - ldb < -MIN_DB)

    rtb = jax.lax.bitcast_convert_type(rt, jnp.int32)
    rtbs = jnp.where(mk, rtb, jnp.int32(0x7FFFFFFF))
    kcnt = jnp.sum(mk.astype(jnp.int32))
    rlo = jnp.maximum((kcnt - 1) // 2, 0)
    rhi = kcnt // 2

    def search_step(_, carry):
        lo, hi = carry
        mid = (lo + hi) // 2
        g = jnp.sum((rtbs <= mid).astype(jnp.int32)) >= rhi + 1
        return jnp.where(g, lo, mid + 1), jnp.where(g, mid, hi)

    theta, _ = jax.lax.fori_loop(
        0, 31, search_step, (jnp.int32(0), jnp.int32(0x7F800000)))
    below = rtbs < theta
    clt = jnp.sum(below.astype(jnp.int32))
    pred = jnp.max(jnp.where(below, rtbs, 0))
    vlo_bits = jnp.where(clt > rlo, pred, theta)
    vlo = jax.lax.bitcast_convert_type(vlo_bits, jnp.float32)
    vhi = jax.lax.bitcast_convert_type(theta, jnp.float32)
    med = (vlo + vhi) * jnp.float32(0.5)

    out = pc_ref[0, 0] + pc_ref[0, 1] * med
    out = jnp.where(kcnt > 0, out, jnp.nan)
    out = jnp.where(jnp.isnan(out), jnp.float32(DEFAULT_RT), out)
    o_ref[...] = jnp.maximum(out, jnp.float32(0.01)).reshape(1, 1, 1)


def kernel(y, poly_coeffs):
    b = y.shape[0]
    t_len = y.shape[-1]
    nf = 1 + t_len // HOP
    pad = K // 2
    y2 = y[:, 0, :]
    off = HOP - pad
    amid = y2[:, off:off + (nf - 1) * HOP].reshape(b, nf - 1, HOP)
    row0 = jnp.concatenate(
        [y2[:, pad:0:-1], y2[:, :off]], axis=1)
    tail_direct = t_len - (nf * HOP - pad)
    npad_r = HOP - tail_direct
    rowl = jnp.concatenate(
        [y2[:, nf * HOP - pad:],
         y2[:, t_len - 2:t_len - 2 - npad_r:-1]], axis=1)
    edges = jnp.stack([row0, rowl], axis=1)
    cs = _dft_mats()
    w1 = cs[:HOP]
    w2 = cs[HOP:]
    pc = poly_coeffs.reshape(1, 2).astype(jnp.float32)

    out = pl.pallas_call(
        functools.partial(_body, nf),
        grid=(b,),
        in_specs=[
            pl.BlockSpec((1, nf - 1, HOP), lambda i: (i, 0, 0)),
            pl.BlockSpec((1, 2, HOP), lambda i: (i, 0, 0)),
            pl.BlockSpec((HOP, 2 * NBINS), lambda i: (0, 0)),
            pl.BlockSpec((K - HOP, 2 * NBINS), lambda i: (0, 0)),
            pl.BlockSpec((1, 2), lambda i: (0, 0)),
        ],
        out_specs=pl.BlockSpec((1, 1, 1), lambda i: (i, 0, 0)),
        out_shape=jax.ShapeDtypeStruct((b, 1, 1), jnp.float32),
        compiler_params=pltpu.CompilerParams(
            dimension_semantics=("arbitrary",)),
    )(amid, edges, w1, w2, pc)
    return out


# speedup vs baseline: 1.6169x; 1.2228x over previous
import functools

import numpy as np
import jax
import jax.numpy as jnp
from jax.experimental import pallas as pl
from jax.experimental.pallas import tpu as pltpu

FS = 16000
M = 800
K = 1024
HOP = 600
NBINS = 256
ORIG_L = 13
MIN_L = 3
MIN_DB = 10.0
DEFAULT_RT = 0.5


def _dft_mats():
    k = np.arange(K, dtype=np.float64)
    w = 0.5 - 0.5 * np.cos(2.0 * np.pi * np.arange(M, dtype=np.float64) / M)
    wf = np.zeros(K, dtype=np.float64)
    lp = (K - M) // 2
    wf[lp:lp + M] = w
    f = np.arange(NBINS, dtype=np.float64)
    ang = 2.0 * np.pi * np.outer(k, f) / K
    cs = np.concatenate([np.cos(ang) * wf[:, None],
                         np.sin(ang) * wf[:, None]], axis=1)
    return jnp.asarray(cs.astype(np.float32))


def _body(nf, nb, a_ref, w1_ref, w2_ref, pc_ref, o_ref):
    w = nb * NBINS
    plist = []
    for j in range(nb):
        a = a_ref[j]
        a0 = a[0:nf, :]
        a1 = a[1:nf + 1, 0:K - HOP]
        cs = (jnp.dot(a0, w1_ref[...], preferred_element_type=jnp.float32) +
              jnp.dot(a1, w2_ref[...], preferred_element_type=jnp.float32))
        c = cs[:, :NBINS]
        s = cs[:, NBINS:]
        plist.append(c * c + s * s)
    P = jnp.concatenate(plist, axis=1)

    Pp = jnp.concatenate(
        [P, jnp.full((ORIG_L, w), jnp.inf, jnp.float32)], axis=0)
    sh = [Pp[t:t + nf, :] for t in range(ORIG_L + 1)]

    dsh = [sh[t + 1] < sh[t] for t in range(ORIG_L - 1)]
    d = dsh[0]

    rl = d.astype(jnp.int32)
    am = d
    for m in range(2, ORIG_L):
        am = am & dsh[m - 1]
        rl = rl + am.astype(jnp.int32)

    maxrl = jnp.max(rl, axis=0, keepdims=True)
    lstar = jnp.where(maxrl >= MIN_L - 1,
                      jnp.minimum(maxrl + 1, ORIG_L), 0)
    valid = (rl + 1 >= lstar) & (lstar >= MIN_L)

    lf = lstar.astype(jnp.float32)
    xm = (lf - 1.0) * 0.5
    den = lf * (lf * lf - 1.0) / 12.0
    acc = jnp.zeros((nf, w), jnp.float32)
    num = jnp.zeros((nf, w), jnp.float32)
    llast = jnp.zeros((nf, w), jnp.float32)
    ldb = None
    for t in range(ORIG_L - 1, -1, -1):
        tm = t < lstar
        acc = acc + jnp.where(tm, sh[t], 0.0)
        ldb = 10.0 * jnp.log10(acc)
        num = num + jnp.where(tm, (t - xm) * ldb, 0.0)
        llast = jnp.where(lstar == t + 1, ldb, llast)
    slope = num / den
    rt = -60.0 / slope * HOP / FS

    mk = valid & (llast - ldb < -MIN_DB)

    rtb = jax.lax.bitcast_convert_type(rt, jnp.int32)
    rtbs = jnp.where(mk, rtb, jnp.int32(0x7FFFFFFF))
    mki = mk.astype(jnp.int32)
    kcnts = [jnp.sum(mki[:, j * NBINS:(j + 1) * NBINS]) for j in range(nb)]
    rlos = [jnp.maximum((k - 1) // 2, 0) for k in kcnts]
    targets = [k // 2 + 1 for k in kcnts]

    def th_row(ths):
        return jnp.concatenate(
            [jnp.full((1, NBINS), t, jnp.int32) for t in ths], axis=1)

    def group_sums(x):
        r = jnp.sum(x, axis=0, keepdims=True)
        return [jnp.sum(r[:, j * NBINS:(j + 1) * NBINS]) for j in range(nb)]

    def search_step(_, carry):
        los, his = carry
        mids = [(lo + hi) // 2 for lo, hi in zip(los, his)]
        cnts = group_sums((rtbs <= th_row(mids)).astype(jnp.int32))
        gs = [c >= t for c, t in zip(cnts, targets)]
        los = tuple(jnp.where(g, lo, mid + 1)
                    for g, lo, mid in zip(gs, los, mids))
        his = tuple(jnp.where(g, mid, hi)
                    for g, mid, hi in zip(gs, mids, his))
        return los, his

    z = jnp.int32(0)
    inf_b = jnp.int32(0x7F800000)
    thetas, _ = jax.lax.fori_loop(
        0, 31, search_step, ((z,) * nb, (inf_b,) * nb))
    below = rtbs < th_row(thetas)
    clts = group_sums(below.astype(jnp.int32))
    bval = jnp.where(below, rtbs, 0)
    bmax = jnp.max(bval, axis=0, keepdims=True)
    outs = []
    for j in range(nb):
        pred = jnp.max(bmax[:, j * NBINS:(j + 1) * NBINS])
        vlo_bits = jnp.where(clts[j] > rlos[j], pred, thetas[j])
        vlo = jax.lax.bitcast_convert_type(vlo_bits, jnp.float32)
        vhi = jax.lax.bitcast_convert_type(thetas[j], jnp.float32)
        med = (vlo + vhi) * jnp.float32(0.5)
        out = pc_ref[0, 0] + pc_ref[0, 1] * med
        out = jnp.where(kcnts[j] > 0, out, jnp.nan)
        out = jnp.where(jnp.isnan(out), jnp.float32(DEFAULT_RT), out)
        outs.append(jnp.maximum(out, jnp.float32(0.01)))
    o_ref[...] = jnp.stack(outs).reshape(nb, 1, 1)


def kernel(y, poly_coeffs):
    b = y.shape[0]
    t_len = y.shape[-1]
    nf = 1 + t_len // HOP
    pad = K // 2
    yp = jnp.pad(y[:, 0, :], ((0, 0), (pad, pad)), mode='reflect')
    nrow = nf + 1
    a = yp[:, :nrow * HOP].reshape(b, nrow, HOP)
    cs = _dft_mats()
    w1 = cs[:HOP]
    w2 = cs[HOP:]
    pc = poly_coeffs.reshape(1, 2).astype(jnp.float32)

    nb = 2 if b % 2 == 0 else 1
    out = pl.pallas_call(
        functools.partial(_body, nf, nb),
        grid=(b // nb,),
        in_specs=[
            pl.BlockSpec((nb, nrow, HOP), lambda i: (i, 0, 0)),
            pl.BlockSpec((HOP, 2 * NBINS), lambda i: (0, 0)),
            pl.BlockSpec((K - HOP, 2 * NBINS), lambda i: (0, 0)),
            pl.BlockSpec((1, 2), lambda i: (0, 0)),
        ],
        out_specs=pl.BlockSpec((nb, 1, 1), lambda i: (i, 0, 0)),
        out_shape=jax.ShapeDtypeStruct((b, 1, 1), jnp.float32),
        compiler_params=pltpu.CompilerParams(
            dimension_semantics=("arbitrary",)),
    )(a, w1, w2, pc)
    return out


# speedup vs baseline: 1.8391x; 1.1374x over previous
import functools

import numpy as np
import jax
import jax.numpy as jnp
from jax.experimental import pallas as pl
from jax.experimental.pallas import tpu as pltpu

FS = 16000
M = 800
K = 1024
HOP = 600
NBINS = 256
ORIG_L = 13
MIN_L = 3
MIN_DB = 10.0
DEFAULT_RT = 0.5


def _dft_mats():
    k = np.arange(K, dtype=np.float64)
    w = 0.5 - 0.5 * np.cos(2.0 * np.pi * np.arange(M, dtype=np.float64) / M)
    wf = np.zeros(K, dtype=np.float64)
    lp = (K - M) // 2
    wf[lp:lp + M] = w
    f = np.arange(NBINS, dtype=np.float64)
    ang = 2.0 * np.pi * np.outer(k, f) / K
    cs = np.concatenate([np.cos(ang) * wf[:, None],
                         np.sin(ang) * wf[:, None]], axis=1)
    return jnp.asarray(cs.astype(np.float32))


def _body(nf, nb, a_ref, w1_ref, w2_ref, pc_ref, o_ref):
    w = nb * NBINS
    plist = []
    for j in range(nb):
        a = a_ref[j]
        a0 = a[0:nf, :]
        a1 = a[1:nf + 1, 0:K - HOP]
        cs = (jnp.dot(a0, w1_ref[...], preferred_element_type=jnp.float32) +
              jnp.dot(a1, w2_ref[...], preferred_element_type=jnp.float32))
        c = cs[:, :NBINS]
        s = cs[:, NBINS:]
        plist.append(c * c + s * s)
    P = jnp.concatenate(plist, axis=1)

    Pp = jnp.concatenate(
        [P, jnp.full((ORIG_L, w), jnp.inf, jnp.float32)], axis=0)
    sh = [Pp[t:t + nf, :] for t in range(ORIG_L + 1)]

    dsh = [sh[t + 1] < sh[t] for t in range(ORIG_L - 1)]
    d = dsh[0]

    rl = d.astype(jnp.int32)
    am = d
    for m in range(2, ORIG_L):
        am = am & dsh[m - 1]
        rl = rl + am.astype(jnp.int32)

    maxrl = jnp.max(rl, axis=0, keepdims=True)
    lstar = jnp.where(maxrl >= MIN_L - 1,
                      jnp.minimum(maxrl + 1, ORIG_L), 0)
    valid = (rl + 1 >= lstar) & (lstar >= MIN_L)

    lf = lstar.astype(jnp.float32)
    xm = (lf - 1.0) * 0.5
    den = lf * (lf * lf - 1.0) / 12.0
    acc = jnp.zeros((nf, w), jnp.float32)
    num = jnp.zeros((nf, w), jnp.float32)
    llast = jnp.zeros((nf, w), jnp.float32)
    ldb = None
    for t in range(ORIG_L - 1, -1, -1):
        tm = t < lstar
        acc = acc + jnp.where(tm, sh[t], 0.0)
        ldb = 10.0 * jnp.log10(acc)
        num = num + jnp.where(tm, (t - xm) * ldb, 0.0)
        llast = jnp.where(lstar == t + 1, ldb, llast)
    slope = num / den
    rt = -60.0 / slope * HOP / FS

    mk = valid & (llast - ldb < -MIN_DB)

    rtb = jax.lax.bitcast_convert_type(rt, jnp.int32)
    rtbs = jnp.where(mk, rtb, jnp.int32(0x7FFFFFFF))
    mki = mk.astype(jnp.int32)
    kcnts = [jnp.sum(mki[:, j * NBINS:(j + 1) * NBINS]) for j in range(nb)]
    rlos = [jnp.maximum((k - 1) // 2, 0) for k in kcnts]
    targets = [k // 2 + 1 for k in kcnts]

    def th_row(ths):
        return jnp.concatenate(
            [jnp.full((1, NBINS), t, jnp.int32) for t in ths], axis=1)

    def group_sums(x):
        r = jnp.sum(x, axis=0, keepdims=True)
        return [jnp.sum(r[:, j * NBINS:(j + 1) * NBINS]) for j in range(nb)]

    def search_step(_, carry):
        los, his = carry
        mids = [(lo + hi) // 2 for lo, hi in zip(los, his)]
        cnts = group_sums((rtbs <= th_row(mids)).astype(jnp.int32))
        gs = [c >= t for c, t in zip(cnts, targets)]
        los = tuple(jnp.where(g, lo, mid + 1)
                    for g, lo, mid in zip(gs, los, mids))
        his = tuple(jnp.where(g, mid, hi)
                    for g, mid, hi in zip(gs, mids, his))
        return los, his

    z = jnp.int32(0)
    inf_b = jnp.int32(0x7F800000)
    thetas, _ = jax.lax.fori_loop(
        0, 31, search_step, ((z,) * nb, (inf_b,) * nb))
    below = rtbs < th_row(thetas)
    clts = group_sums(below.astype(jnp.int32))
    bval = jnp.where(below, rtbs, 0)
    bmax = jnp.max(bval, axis=0, keepdims=True)
    outs = []
    for j in range(nb):
        pred = jnp.max(bmax[:, j * NBINS:(j + 1) * NBINS])
        vlo_bits = jnp.where(clts[j] > rlos[j], pred, thetas[j])
        vlo = jax.lax.bitcast_convert_type(vlo_bits, jnp.float32)
        vhi = jax.lax.bitcast_convert_type(thetas[j], jnp.float32)
        med = (vlo + vhi) * jnp.float32(0.5)
        out = pc_ref[0, 0] + pc_ref[0, 1] * med
        out = jnp.where(kcnts[j] > 0, out, jnp.nan)
        out = jnp.where(jnp.isnan(out), jnp.float32(DEFAULT_RT), out)
        outs.append(jnp.maximum(out, jnp.float32(0.01)))
    o_ref[...] = jnp.stack(outs).reshape(nb, 1, 1)


def kernel(y, poly_coeffs):
    b = y.shape[0]
    t_len = y.shape[-1]
    nf = 1 + t_len // HOP
    pad = K // 2
    yp = jnp.pad(y[:, 0, :], ((0, 0), (pad, pad)), mode='reflect')
    nrow = nf + 1
    a = yp[:, :nrow * HOP].reshape(b, nrow, HOP)
    cs = _dft_mats()
    w1 = cs[:HOP]
    w2 = cs[HOP:]
    pc = poly_coeffs.reshape(1, 2).astype(jnp.float32)

    nb = 4 if b % 4 == 0 else (2 if b % 2 == 0 else 1)
    out = pl.pallas_call(
        functools.partial(_body, nf, nb),
        grid=(b // nb,),
        in_specs=[
            pl.BlockSpec((nb, nrow, HOP), lambda i: (i, 0, 0)),
            pl.BlockSpec((HOP, 2 * NBINS), lambda i: (0, 0)),
            pl.BlockSpec((K - HOP, 2 * NBINS), lambda i: (0, 0)),
            pl.BlockSpec((1, 2), lambda i: (0, 0)),
        ],
        out_specs=pl.BlockSpec((nb, 1, 1), lambda i: (i, 0, 0)),
        out_shape=jax.ShapeDtypeStruct((b, 1, 1), jnp.float32),
        compiler_params=pltpu.CompilerParams(
            dimension_semantics=("arbitrary",)),
    )(a, w1, w2, pc)
    return out
